# baseline (speedup 1.0000x reference)
import functools

import jax
import jax.numpy as jnp
from jax import lax
from jax.experimental import pallas as pl
from jax.experimental.pallas import tpu as pltpu
from jax.experimental.pallas import tpu_sc as plsc

N = 32768
D = 128
A = 64
B = 16
L = 16
NC = 2
NS = 16
NW = NC * NS
RPW = N // NW
BLK = 256
NBLK = RPW // BLK
DC = D // L

_mesh = plsc.VectorSubcoreMesh(
    core_axis_name="c", subcore_axis_name="s", num_cores=NC, num_subcores=NS
)
_params = pltpu.CompilerParams(needs_layout_passes=False)


def _wid():
    return lax.axis_index("s") * NC + lax.axis_index("c")


def _hreduce(vec, op):
    vals = [vec[j] for j in range(L)]
    while len(vals) > 1:
        vals = [op(vals[i], vals[i + 1]) for i in range(0, len(vals), 2)]
    return vals[0]


@functools.partial(
    pl.kernel,
    out_type=(
        jax.ShapeDtypeStruct((NW, B * D), jnp.float32),
        jax.ShapeDtypeStruct((NW, L), jnp.float32),
        jax.ShapeDtypeStruct((NW, L), jnp.float32),
    ),
    mesh=_mesh,
    compiler_params=_params,
    scratch_types=[
        pltpu.VMEM((2, BLK * D), jnp.float32),
        pltpu.VMEM((RPW,), jnp.float32),
        pltpu.VMEM((RPW,), jnp.int32),
        pltpu.VMEM((A, D), jnp.float32),
        pltpu.VMEM((A,), jnp.float32),
        pltpu.VMEM((A,), jnp.float32),
        pltpu.VMEM((B * D,), jnp.float32),
        pltpu.VMEM((L,), jnp.float32),
        pltpu.VMEM((L,), jnp.float32),
        pltpu.SemaphoreType.DMA,
        pltpu.SemaphoreType.DMA,
    ],
)
def _main_k(x_hbm, seg_hbm, w_hbm, bias_hbm, v_hbm,
            accp_hbm, denp_hbm, maxw_hbm,
            xb_v, sc_v, sg_v, w_v, v_v, bias_v, acc_v, den_v, mst_v,
            sem0, sem1):
    wid = _wid()
    row0 = wid * RPW
    sems = [sem0, sem1]

    copies = [None] * NBLK
    copies[0] = pltpu.async_copy(
        x_hbm.at[pl.ds(row0 * D, BLK * D)], xb_v.at[0], sems[0])

    pltpu.sync_copy(w_hbm, w_v)
    pltpu.sync_copy(v_hbm, v_v)
    pltpu.sync_copy(bias_hbm, bias_v)
    pltpu.sync_copy(seg_hbm.at[pl.ds(row0, RPW)], sg_v)

    vch = [v_v[i * L:(i + 1) * L] for i in range(A // L)]
    bch = [bias_v[i * L:(i + 1) * L] for i in range(A // L)]
    cvec = bch[0] * vch[0]
    for i in range(1, A // L):
        cvec = cvec + bch[i] * vch[i]
    c = _hreduce(cvec, jnp.add)

    uc = [jnp.zeros((L,), jnp.float32) for _ in range(DC)]
    for a in range(A):
        va = vch[a // L][a % L]
        for k in range(DC):
            uc[k] = uc[k] + w_v[a, k * L:(k + 1) * L] * va

    lane = jnp.arange(L, dtype=jnp.int32)

    den_v[...] = jnp.zeros((L,), jnp.float32)
    for k in range(B * D // L):
        acc_v[k * L:(k + 1) * L] = jnp.zeros((L,), jnp.float32)

    m = jnp.float32(-1e30)

    for blk in range(NBLK):
        bi = blk % 2
        if blk + 1 < NBLK:
            copies[blk + 1] = pltpu.async_copy(
                x_hbm.at[pl.ds((row0 + (blk + 1) * BLK) * D, BLK * D)],
                xb_v.at[(blk + 1) % 2], sems[(blk + 1) % 2])
        copies[blk].wait()

        def grp_body(g2, bmv, blk=blk, bi=bi):
            for gg in range(2):
                g = g2 * 2 + gg
                base = g * (L * D)
                svals = []
                for j in range(L):
                    rb = base + j * D
                    ch = [xb_v[bi, pl.ds(rb + k * L, L)] * uc[k]
                          for k in range(DC)]
                    acc = ((ch[0] + ch[1]) + (ch[2] + ch[3])) + \
                          ((ch[4] + ch[5]) + (ch[6] + ch[7]))
                    svals.append(jnp.sum(acc))
                s = jnp.zeros((L,), jnp.float32)
                for j in range(L):
                    s = jnp.where(lane == j, svals[j], s)
                s = s + c
                sc_v[pl.ds(blk * BLK + g * L, L)] = s
                bmv = jnp.maximum(bmv, s)
            return bmv

        bmv = lax.fori_loop(
            0, BLK // L // 2, grp_body, jnp.full((L,), -1e30, jnp.float32))
        m_new = jnp.maximum(m, _hreduce(bmv, jnp.maximum))

        def rescale(m=m, m_new=m_new):
            scv = jnp.exp(jnp.full((L,), m - m_new, jnp.float32))
            den_v[...] = den_v[0:L] * scv
            for k in range(B * D // L):
                acc_v[k * L:(k + 1) * L] = acc_v[k * L:(k + 1) * L] * scv

        lax.cond(m_new > m, rescale, lambda: None)
        m = m_new

        def acc_body(g2, _, blk=blk, bi=bi, m=m):
            for gg in range(2):
                g = g2 * 2 + gg
                gbase = blk * BLK + g * L
                e16 = jnp.exp(sc_v[pl.ds(gbase, L)] - m)
                s16 = sg_v[pl.ds(gbase, L)]
                plsc.addupdate_scatter(den_v, [s16], e16)
                xbase = g * (L * D)

                def same_seg(e16=e16, s16=s16, xbase=xbase):
                    accs = [jnp.zeros((L,), jnp.float32) for _ in range(DC)]
                    for j in range(L):
                        er = e16[j]
                        for k in range(DC):
                            accs[k] = accs[k] + \
                                xb_v[bi, pl.ds(xbase + j * D + k * L, L)] * er
                    sb = s16[0] * D
                    for k in range(DC):
                        plsc.addupdate(
                            acc_v.at[pl.ds(sb + k * L, L)], accs[k])

                def mixed_seg(e16=e16, s16=s16, xbase=xbase):
                    for j in range(L):
                        er = e16[j]
                        sb = s16[j] * D
                        for k in range(DC):
                            plsc.addupdate(
                                acc_v.at[pl.ds(sb + k * L, L)],
                                xb_v[bi, pl.ds(xbase + j * D + k * L, L)]
                                * er)

                lax.cond(s16[0] == s16[L - 1], same_seg, mixed_seg)
            return 0

        lax.fori_loop(0, BLK // L // 2, acc_body, 0)

    mst_v[...] = jnp.full((L,), m, jnp.float32)
    pltpu.sync_copy(acc_v, accp_hbm.at[wid])
    pltpu.sync_copy(den_v, denp_hbm.at[wid])
    pltpu.sync_copy(mst_v, maxw_hbm.at[wid])


@functools.partial(
    pl.kernel,
    out_type=jax.ShapeDtypeStruct((B, D), jnp.float32),
    mesh=_mesh,
    compiler_params=_params,
    scratch_types=[
        pltpu.VMEM((NW, D), jnp.float32),
        pltpu.VMEM((NW, L), jnp.float32),
        pltpu.VMEM((NW, L), jnp.float32),
        pltpu.VMEM((L,), jnp.float32),
        pltpu.VMEM((D,), jnp.float32),
    ],
)
def _comb_k(accp_hbm, denp_hbm, maxw_hbm, out_hbm,
            accf_v, den_v, mx_v, dst_v, ost_v):
    wid = _wid()

    @pl.when(wid < B)
    def _():
        pltpu.sync_copy(accp_hbm.at[:, pl.ds(wid * D, D)], accf_v)
        pltpu.sync_copy(denp_hbm, den_v)
        pltpu.sync_copy(maxw_hbm, mx_v)

        gmaxv = mx_v[0, 0:L]
        for w2 in range(1, NW):
            gmaxv = jnp.maximum(gmaxv, mx_v[w2, 0:L])
        fv = [jnp.exp(mx_v[w2, 0:L] - gmaxv) for w2 in range(NW)]

        dv = den_v[0, 0:L] * fv[0]
        for w2 in range(1, NW):
            dv = dv + den_v[w2, 0:L] * fv[w2]
        dst_v[...] = dv
        dbv = plsc.load_gather(dst_v, [jnp.full((L,), wid, jnp.int32)])
        rdenv = jnp.where(dbv > 0.0, 1.0 / dbv, 0.0)

        for k in range(DC):
            ak = accf_v[0, k * L:(k + 1) * L] * fv[0]
            for w2 in range(1, NW):
                ak = ak + accf_v[w2, k * L:(k + 1) * L] * fv[w2]
            ost_v[k * L:(k + 1) * L] = ak * rdenv

        pltpu.sync_copy(ost_v, out_hbm.at[wid])


def kernel(x, residue_mask, W, b, v):
    xf = x.reshape(-1)
    seg = residue_mask.astype(jnp.int32)
    accp, denp, maxw = _main_k(xf, seg, W, b, v)
    return _comb_k(accp, denp, maxw)


# speedup vs baseline: 1.1477x; 1.1477x over previous
import functools

import jax
import jax.numpy as jnp
from jax import lax
from jax.experimental import pallas as pl
from jax.experimental.pallas import tpu as pltpu
from jax.experimental.pallas import tpu_sc as plsc

N = 32768
D = 128
A = 64
B = 16
L = 16
NC = 2
NS = 16
NW = NC * NS
RPW = N // NW
BLK = 256
NBLK = RPW // BLK
DC = D // L

_mesh = plsc.VectorSubcoreMesh(
    core_axis_name="c", subcore_axis_name="s", num_cores=NC, num_subcores=NS
)
_params = pltpu.CompilerParams(needs_layout_passes=False)


def _wid():
    return lax.axis_index("s") * NC + lax.axis_index("c")


def _hreduce(vec, op):
    vals = [vec[j] for j in range(L)]
    while len(vals) > 1:
        vals = [op(vals[i], vals[i + 1]) for i in range(0, len(vals), 2)]
    return vals[0]


@functools.partial(
    pl.kernel,
    out_type=(
        jax.ShapeDtypeStruct((NW, B * D), jnp.float32),
        jax.ShapeDtypeStruct((NW, L), jnp.float32),
        jax.ShapeDtypeStruct((NW, L), jnp.float32),
    ),
    mesh=_mesh,
    compiler_params=_params,
    scratch_types=[
        pltpu.VMEM((2, BLK * D), jnp.float32),
        pltpu.VMEM((RPW,), jnp.float32),
        pltpu.VMEM((RPW,), jnp.int32),
        pltpu.VMEM((A, D), jnp.float32),
        pltpu.VMEM((A,), jnp.float32),
        pltpu.VMEM((A,), jnp.float32),
        pltpu.VMEM((B * D,), jnp.float32),
        pltpu.VMEM((L,), jnp.float32),
        pltpu.VMEM((L,), jnp.float32),
        pltpu.SemaphoreType.DMA,
        pltpu.SemaphoreType.DMA,
    ],
)
def _main_k(x_hbm, seg_hbm, w_hbm, bias_hbm, v_hbm,
            accp_hbm, denp_hbm, maxw_hbm,
            xb_v, sc_v, sg_v, w_v, v_v, bias_v, acc_v, den_v, mst_v,
            sem0, sem1):
    wid = _wid()
    row0 = wid * RPW
    sems = [sem0, sem1]

    copies = [None] * NBLK
    copies[0] = pltpu.async_copy(
        x_hbm.at[pl.ds(row0 * D, BLK * D)], xb_v.at[0], sems[0])

    pltpu.sync_copy(w_hbm, w_v)
    pltpu.sync_copy(v_hbm, v_v)
    pltpu.sync_copy(bias_hbm, bias_v)
    pltpu.sync_copy(seg_hbm.at[pl.ds(row0, RPW)], sg_v)

    vch = [v_v[i * L:(i + 1) * L] for i in range(A // L)]
    bch = [bias_v[i * L:(i + 1) * L] for i in range(A // L)]
    cvec = bch[0] * vch[0]
    for i in range(1, A // L):
        cvec = cvec + bch[i] * vch[i]
    c = _hreduce(cvec, jnp.add)

    uc = [jnp.zeros((L,), jnp.float32) for _ in range(DC)]
    for a in range(A):
        va = vch[a // L][a % L]
        for k in range(DC):
            uc[k] = uc[k] + w_v[a, k * L:(k + 1) * L] * va

    lane = jnp.arange(L, dtype=jnp.int32)

    den_v[...] = jnp.zeros((L,), jnp.float32)
    for k in range(B * D // L):
        acc_v[k * L:(k + 1) * L] = jnp.zeros((L,), jnp.float32)

    m = jnp.float32(-1e30)

    for blk in range(NBLK):
        bi = blk % 2
        if blk + 1 < NBLK:
            copies[blk + 1] = pltpu.async_copy(
                x_hbm.at[pl.ds((row0 + (blk + 1) * BLK) * D, BLK * D)],
                xb_v.at[(blk + 1) % 2], sems[(blk + 1) % 2])
        copies[blk].wait()

        def grp_body(g, bmv, blk=blk, bi=bi):
            base = g * (L * D)
            svals = []
            for j in range(L):
                rb = base + j * D
                ch = [xb_v[bi, pl.ds(rb + k * L, L)] * uc[k]
                      for k in range(DC)]
                acc = ((ch[0] + ch[1]) + (ch[2] + ch[3])) + \
                      ((ch[4] + ch[5]) + (ch[6] + ch[7]))
                svals.append(jnp.sum(acc))
            s = jnp.zeros((L,), jnp.float32)
            for j in range(L):
                s = jnp.where(lane == j, svals[j], s)
            s = s + c
            sc_v[pl.ds(blk * BLK + g * L, L)] = s
            return jnp.maximum(bmv, s)

        bmv = lax.fori_loop(
            0, BLK // L, grp_body, jnp.full((L,), -1e30, jnp.float32))
        m_new = jnp.maximum(m, _hreduce(bmv, jnp.maximum))

        def rescale(m=m, m_new=m_new):
            scv = jnp.exp(jnp.full((L,), m - m_new, jnp.float32))
            den_v[...] = den_v[0:L] * scv
            for k in range(B * D // L):
                acc_v[k * L:(k + 1) * L] = acc_v[k * L:(k + 1) * L] * scv

        lax.cond(m_new > m, rescale, lambda: None)
        m = m_new

        def acc_body(g, _, blk=blk, bi=bi, m=m):
            gbase = blk * BLK + g * L
            e16 = jnp.exp(sc_v[pl.ds(gbase, L)] - m)
            s16 = sg_v[pl.ds(gbase, L)]
            plsc.addupdate_scatter(den_v, [s16], e16)
            xbase = g * (L * D)

            def same_seg():
                accs = [jnp.zeros((L,), jnp.float32) for _ in range(DC)]
                for j in range(L):
                    er = e16[j]
                    for k in range(DC):
                        accs[k] = accs[k] + \
                            xb_v[bi, pl.ds(xbase + j * D + k * L, L)] * er
                sb = s16[0] * D
                for k in range(DC):
                    plsc.addupdate(acc_v.at[pl.ds(sb + k * L, L)], accs[k])

            def mixed_seg():
                for j in range(L):
                    er = e16[j]
                    sb = s16[j] * D
                    for k in range(DC):
                        plsc.addupdate(
                            acc_v.at[pl.ds(sb + k * L, L)],
                            xb_v[bi, pl.ds(xbase + j * D + k * L, L)] * er)

            lax.cond(s16[0] == s16[L - 1], same_seg, mixed_seg)
            return 0

        lax.fori_loop(0, BLK // L, acc_body, 0)

    mst_v[...] = jnp.full((L,), m, jnp.float32)
    pltpu.sync_copy(acc_v, accp_hbm.at[wid])
    pltpu.sync_copy(den_v, denp_hbm.at[wid])
    pltpu.sync_copy(mst_v, maxw_hbm.at[wid])


def _comb_tc_body(accp_ref, denp_ref, maxw_ref, out_ref):
    maxw = maxw_ref[...]
    gmax = jnp.max(maxw)
    f = jnp.exp(maxw - gmax)
    den = jnp.sum(denp_ref[...] * f, axis=0)
    acc = accp_ref[...].reshape(NW, B, D)
    num = jnp.sum(acc * f[:, :, None], axis=0)
    rden = jnp.where(den > 0.0, 1.0 / den, 0.0)
    out_ref[...] = num * rden[:, None]


_comb_tc = pl.pallas_call(
    _comb_tc_body,
    out_shape=jax.ShapeDtypeStruct((B, D), jnp.float32),
)


def kernel(x, residue_mask, W, b, v):
    xf = x.reshape(-1)
    seg = residue_mask.astype(jnp.int32)
    accp, denp, maxw = _main_k(xf, seg, W, b, v)
    return _comb_tc(accp, denp, maxw)


# speedup vs baseline: 1.1745x; 1.0234x over previous
import functools

import jax
import jax.numpy as jnp
from jax import lax
from jax.experimental import pallas as pl
from jax.experimental.pallas import tpu as pltpu
from jax.experimental.pallas import tpu_sc as plsc

N = 32768
D = 128
A = 64
B = 16
L = 16
NC = 2
NS = 16
NW = NC * NS
RPW = N // NW
BLK = 256
NBLK = RPW // BLK
DC = D // L

_mesh = plsc.VectorSubcoreMesh(
    core_axis_name="c", subcore_axis_name="s", num_cores=NC, num_subcores=NS
)
_params = pltpu.CompilerParams(needs_layout_passes=False)


def _wid():
    return lax.axis_index("s") * NC + lax.axis_index("c")


def _hreduce(vec, op):
    vals = [vec[j] for j in range(L)]
    while len(vals) > 1:
        vals = [op(vals[i], vals[i + 1]) for i in range(0, len(vals), 2)]
    return vals[0]


@functools.partial(
    pl.kernel,
    out_type=(
        jax.ShapeDtypeStruct((NW, B * D), jnp.float32),
        jax.ShapeDtypeStruct((NW, L), jnp.float32),
        jax.ShapeDtypeStruct((NW, L), jnp.float32),
    ),
    mesh=_mesh,
    compiler_params=_params,
    scratch_types=[
        pltpu.VMEM((2, BLK * D), jnp.float32),
        pltpu.VMEM((RPW,), jnp.float32),
        pltpu.VMEM((RPW,), jnp.int32),
        pltpu.VMEM((A, D), jnp.float32),
        pltpu.VMEM((A,), jnp.float32),
        pltpu.VMEM((A,), jnp.float32),
        pltpu.VMEM((B * D,), jnp.float32),
        pltpu.VMEM((L,), jnp.float32),
        pltpu.VMEM((L,), jnp.float32),
        pltpu.SemaphoreType.DMA,
        pltpu.SemaphoreType.DMA,
        pltpu.SemaphoreType.DMA,
    ],
)
def _main_k(x_hbm, seg_hbm, w_hbm, bias_hbm, v_hbm,
            accp_hbm, denp_hbm, maxw_hbm,
            xb_v, sc_v, sg_v, w_v, v_v, bias_v, acc_v, den_v, mst_v,
            sem0, sem1, sem2):
    wid = _wid()
    row0 = wid * RPW
    sems = [sem0, sem1]

    copies = [None] * NBLK
    copies[0] = pltpu.async_copy(
        x_hbm.at[pl.ds(row0 * D, BLK * D)], xb_v.at[0], sems[0])

    cw = pltpu.async_copy(w_hbm, w_v, sem2)
    cv = pltpu.async_copy(v_hbm, v_v, sem2)
    cb = pltpu.async_copy(bias_hbm, bias_v, sem2)
    cs = pltpu.async_copy(seg_hbm.at[pl.ds(row0, RPW)], sg_v, sem2)
    cw.wait()
    cv.wait()
    cb.wait()

    vch = [v_v[i * L:(i + 1) * L] for i in range(A // L)]
    bch = [bias_v[i * L:(i + 1) * L] for i in range(A // L)]
    cvec = bch[0] * vch[0]
    for i in range(1, A // L):
        cvec = cvec + bch[i] * vch[i]
    c = _hreduce(cvec, jnp.add)

    uc = [jnp.zeros((L,), jnp.float32) for _ in range(DC)]
    for a in range(A):
        va = vch[a // L][a % L]
        for k in range(DC):
            uc[k] = uc[k] + w_v[a, k * L:(k + 1) * L] * va

    lane = jnp.arange(L, dtype=jnp.int32)
    cs.wait()

    den_v[...] = jnp.zeros((L,), jnp.float32)
    for k in range(B * D // L):
        acc_v[k * L:(k + 1) * L] = jnp.zeros((L,), jnp.float32)

    m = jnp.float32(-1e30)

    for blk in range(NBLK):
        bi = blk % 2
        if blk + 1 < NBLK:
            copies[blk + 1] = pltpu.async_copy(
                x_hbm.at[pl.ds((row0 + (blk + 1) * BLK) * D, BLK * D)],
                xb_v.at[(blk + 1) % 2], sems[(blk + 1) % 2])
        copies[blk].wait()

        def grp_body(g, bmv, blk=blk, bi=bi):
            base = g * (L * D)
            svals = []
            for j in range(L):
                rb = base + j * D
                ch = [xb_v[bi, pl.ds(rb + k * L, L)] * uc[k]
                      for k in range(DC)]
                acc = ((ch[0] + ch[1]) + (ch[2] + ch[3])) + \
                      ((ch[4] + ch[5]) + (ch[6] + ch[7]))
                svals.append(jnp.sum(acc))
            s = jnp.zeros((L,), jnp.float32)
            for j in range(L):
                s = jnp.where(lane == j, svals[j], s)
            s = s + c
            sc_v[pl.ds(blk * BLK + g * L, L)] = s
            return jnp.maximum(bmv, s)

        bmv = lax.fori_loop(
            0, BLK // L, grp_body, jnp.full((L,), -1e30, jnp.float32))
        m_new = jnp.maximum(m, _hreduce(bmv, jnp.maximum))

        def rescale(m=m, m_new=m_new):
            scv = jnp.exp(jnp.full((L,), m - m_new, jnp.float32))
            den_v[...] = den_v[0:L] * scv
            for k in range(B * D // L):
                acc_v[k * L:(k + 1) * L] = acc_v[k * L:(k + 1) * L] * scv

        lax.cond(m_new > m, rescale, lambda: None)
        m = m_new

        def acc_body(g, _, blk=blk, bi=bi, m=m):
            gbase = blk * BLK + g * L
            e16 = jnp.exp(sc_v[pl.ds(gbase, L)] - m)
            s16 = sg_v[pl.ds(gbase, L)]
            plsc.addupdate_scatter(den_v, [s16], e16)
            xbase = g * (L * D)

            def same_seg():
                accs = [jnp.zeros((L,), jnp.float32) for _ in range(DC)]
                for j in range(L):
                    er = e16[j]
                    for k in range(DC):
                        accs[k] = accs[k] + \
                            xb_v[bi, pl.ds(xbase + j * D + k * L, L)] * er
                sb = s16[0] * D
                for k in range(DC):
                    plsc.addupdate(acc_v.at[pl.ds(sb + k * L, L)], accs[k])

            def mixed_seg():
                for j in range(L):
                    er = e16[j]
                    sb = s16[j] * D
                    for k in range(DC):
                        plsc.addupdate(
                            acc_v.at[pl.ds(sb + k * L, L)],
                            xb_v[bi, pl.ds(xbase + j * D + k * L, L)] * er)

            lax.cond(s16[0] == s16[L - 1], same_seg, mixed_seg)
            return 0

        lax.fori_loop(0, BLK // L, acc_body, 0)

    mst_v[...] = jnp.full((L,), m, jnp.float32)
    pltpu.sync_copy(acc_v, accp_hbm.at[wid])
    pltpu.sync_copy(den_v, denp_hbm.at[wid])
    pltpu.sync_copy(mst_v, maxw_hbm.at[wid])


def _comb_tc_body(accp_ref, denp_ref, maxw_ref, out_ref):
    maxw = maxw_ref[...]
    gmax = jnp.max(maxw)
    f = jnp.exp(maxw - gmax)
    den = jnp.sum(denp_ref[...] * f, axis=0)
    acc = accp_ref[...].reshape(NW, B, D)
    num = jnp.sum(acc * f[:, :, None], axis=0)
    rden = jnp.where(den > 0.0, 1.0 / den, 0.0)
    out_ref[...] = num * rden[:, None]


_comb_tc = pl.pallas_call(
    _comb_tc_body,
    out_shape=jax.ShapeDtypeStruct((B, D), jnp.float32),
)


def kernel(x, residue_mask, W, b, v):
    xf = x.reshape(-1)
    seg = residue_mask.astype(jnp.int32)
    accp, denp, maxw = _main_k(xf, seg, W, b, v)
    return _comb_tc(accp, denp, maxw)
